# Initial kernel scaffold; baseline (speedup 1.0000x reference)
import jax
import jax.numpy as jnp
from jax.experimental import pallas as pl


def kernel(src, frac, cbfv, W, b, emb_scaler, pos_scaler, pos_scaler_log):
    raise NotImplementedError("write your pallas kernel here")


# speedup vs baseline: 1.0445x; 1.0445x over previous
import functools

import jax
import jax.numpy as jnp
from jax import lax
from jax.experimental import pallas as pl
from jax.experimental.pallas import tpu as pltpu
from jax.experimental.pallas import tpu_sc as plsc

D_MODEL = 512
HALF = D_MODEL // 2
RES = 5000
FEAT = 200
VOCAB = 119
B, T = 4096, 20
N = B * T

_NC, _NS = 2, 16
NW = _NC * _NS
TOK_PER_W = N // NW
CHUNK = 64
N_CHUNKS = TOK_PER_W // CHUNK

_VPAD = 120
_KPAD = 256


def _pe_table():
    x = jnp.arange(RES, dtype=jnp.float32).reshape(RES, 1)
    fr = jnp.tile(jnp.arange(HALF, dtype=jnp.float32).reshape(1, HALF), (RES, 1))
    pe = jnp.zeros((RES, HALF), dtype=jnp.float32)
    pe = pe.at[:, 0::2].set(jnp.sin(x / jnp.power(50.0, 2.0 * fr[:, 0::2] / HALF)))
    pe = pe.at[:, 1::2].set(jnp.cos(x / jnp.power(50.0, 2.0 * fr[:, 1::2] / HALF)))
    return pe


def _prep(frac_ref, cbfv_ref, w_ref, brow_ref, es_ref, ps_ref, pls_ref, pe_ref,
          v2_ref, il_ref, ig_ref, p1_ref, p2_ref):
    x = jnp.dot(cbfv_ref[...], w_ref[...], preferred_element_type=jnp.float32)
    v2_ref[...] = (x + brow_ref[...]) * (2.0 ** es_ref[...])
    p1_ref[...] = pe_ref[...] * (2.0 ** ((1.0 - ps_ref[...]) ** 2))
    p2_ref[...] = pe_ref[...] * (2.0 ** ((1.0 - pls_ref[...]) ** 2))
    f = frac_ref[...]
    xf = jnp.maximum(f, 1.0 / RES)
    il_ref[...] = jnp.round(xf * RES).astype(jnp.int32) - 1
    lg = jnp.log2(f)
    xl = 0.0025 * lg * lg
    xl = jnp.maximum(jnp.minimum(xl, 1.0), 1.0 / RES)
    ig_ref[...] = jnp.round(xl * RES).astype(jnp.int32) - 1


def _sc_body(v2_hbm, p1_hbm, p2_hbm, src_hbm, il_hbm, ig_hbm, out_hbm,
             srcv, ilv, igv, buf, pa, pb, s0, s1, s2):
    wid = lax.axis_index("s") * _NC + lax.axis_index("c")

    def chunk_body(k, carry):
        base = wid * TOK_PER_W + k * CHUNK
        pltpu.sync_copy(src_hbm.at[pl.ds(base, CHUNK)], srcv)
        pltpu.sync_copy(il_hbm.at[pl.ds(base, CHUNK)], ilv)
        pltpu.sync_copy(ig_hbm.at[pl.ds(base, CHUNK)], igv)
        cp0 = pltpu.async_copy(v2_hbm.at[srcv], buf, s0)
        cp1 = pltpu.async_copy(p1_hbm.at[ilv], pa, s1)
        cp2 = pltpu.async_copy(p2_hbm.at[igv], pb, s2)
        cp0.wait()
        cp1.wait()
        cp2.wait()

        def tok_body(t, c):
            for j in range(HALF // 16):
                plsc.addupdate(buf.at[t, pl.ds(j * 16, 16)],
                               pa[t, pl.ds(j * 16, 16)])
            for j in range(HALF // 16):
                plsc.addupdate(buf.at[t, pl.ds(HALF + j * 16, 16)],
                               pb[t, pl.ds(j * 16, 16)])
            return c

        lax.fori_loop(0, CHUNK, tok_body, 0)
        pltpu.sync_copy(buf, out_hbm.at[pl.ds(base, CHUNK)])
        return carry

    lax.fori_loop(0, N_CHUNKS, chunk_body, 0)


def kernel(src, frac, cbfv, W, b, emb_scaler, pos_scaler, pos_scaler_log):
    pe = _pe_table()
    frac2 = frac.reshape(N // 128, 128)
    cbfv_pad = jnp.zeros((_VPAD, _KPAD), jnp.float32).at[:VOCAB, :FEAT].set(cbfv)
    w_pad = jnp.zeros((_KPAD, D_MODEL), jnp.float32).at[:FEAT, :].set(W)
    brow = b.reshape(1, D_MODEL)
    es = jnp.broadcast_to(emb_scaler.reshape(1, 1), (1, D_MODEL))
    ps = jnp.broadcast_to(pos_scaler.reshape(1, 1), (1, HALF))
    pls = jnp.broadcast_to(pos_scaler_log.reshape(1, 1), (1, HALF))

    v2, il, ig, p1, p2 = pl.pallas_call(
        _prep,
        out_shape=[
            jax.ShapeDtypeStruct((_VPAD, D_MODEL), jnp.float32),
            jax.ShapeDtypeStruct((N // 128, 128), jnp.int32),
            jax.ShapeDtypeStruct((N // 128, 128), jnp.int32),
            jax.ShapeDtypeStruct((RES, HALF), jnp.float32),
            jax.ShapeDtypeStruct((RES, HALF), jnp.float32),
        ],
    )(frac2, cbfv_pad, w_pad, brow, es, ps, pls, pe)

    mesh = plsc.VectorSubcoreMesh(core_axis_name="c", subcore_axis_name="s")
    sc = functools.partial(
        pl.kernel,
        mesh=mesh,
        out_type=jax.ShapeDtypeStruct((N, D_MODEL), jnp.float32),
        scratch_types=[
            pltpu.VMEM((CHUNK,), jnp.int32),
            pltpu.VMEM((CHUNK,), jnp.int32),
            pltpu.VMEM((CHUNK,), jnp.int32),
            pltpu.VMEM((CHUNK, D_MODEL), jnp.float32),
            pltpu.VMEM((CHUNK, HALF), jnp.float32),
            pltpu.VMEM((CHUNK, HALF), jnp.float32),
            pltpu.SemaphoreType.DMA,
            pltpu.SemaphoreType.DMA,
            pltpu.SemaphoreType.DMA,
        ],
    )(_sc_body)

    out = sc(v2, p1, p2, src.reshape(N), il.reshape(N), ig.reshape(N))
    return out.reshape(B, T, D_MODEL)


# speedup vs baseline: 1.0982x; 1.0515x over previous
import functools

import jax
import jax.numpy as jnp
from jax import lax
from jax.experimental import pallas as pl
from jax.experimental.pallas import tpu as pltpu
from jax.experimental.pallas import tpu_sc as plsc

D_MODEL = 512
HALF = D_MODEL // 2
RES = 5000
FEAT = 200
VOCAB = 119
B, T = 4096, 20
N = B * T

_NC, _NS = 2, 16
NW = _NC * _NS
TOK_PER_W = N // NW
CHUNK = 40
N_CHUNKS = TOK_PER_W // CHUNK

_VPAD = 120
_KPAD = 256


def _pe_table():
    x = jnp.arange(RES, dtype=jnp.float32).reshape(RES, 1)
    fr = jnp.tile(jnp.arange(HALF, dtype=jnp.float32).reshape(1, HALF), (RES, 1))
    pe = jnp.zeros((RES, HALF), dtype=jnp.float32)
    pe = pe.at[:, 0::2].set(jnp.sin(x / jnp.power(50.0, 2.0 * fr[:, 0::2] / HALF)))
    pe = pe.at[:, 1::2].set(jnp.cos(x / jnp.power(50.0, 2.0 * fr[:, 1::2] / HALF)))
    return pe


def _prep(frac_ref, cbfv_ref, w_ref, brow_ref, es_ref, ps_ref, pls_ref, pe_ref,
          v2_ref, il_ref, ig_ref, p1_ref, p2_ref):
    x = jnp.dot(cbfv_ref[...], w_ref[...], preferred_element_type=jnp.float32)
    v2_ref[...] = (x + brow_ref[...]) * (2.0 ** es_ref[...])
    p1_ref[...] = pe_ref[...] * (2.0 ** ((1.0 - ps_ref[...]) ** 2))
    p2_ref[...] = pe_ref[...] * (2.0 ** ((1.0 - pls_ref[...]) ** 2))
    f = frac_ref[...]
    xf = jnp.maximum(f, 1.0 / RES)
    il_ref[...] = jnp.round(xf * RES).astype(jnp.int32) - 1
    lg = jnp.log2(f)
    xl = 0.0025 * lg * lg
    xl = jnp.maximum(jnp.minimum(xl, 1.0), 1.0 / RES)
    ig_ref[...] = jnp.round(xl * RES).astype(jnp.int32) - 1


def _sc_body(v2_hbm, p1_hbm, p2_hbm, src_hbm, il_hbm, ig_hbm, out_hbm,
             srcv, ilv, igv, buf0, pa0, pb0, buf1, pa1, pb1,
             sg0, sg1, so0, so1):
    wid = lax.axis_index("s") * _NC + lax.axis_index("c")

    pltpu.sync_copy(src_hbm.at[pl.ds(wid * N_CHUNKS, N_CHUNKS)], srcv)
    pltpu.sync_copy(il_hbm.at[pl.ds(wid * N_CHUNKS, N_CHUNKS)], ilv)
    pltpu.sync_copy(ig_hbm.at[pl.ds(wid * N_CHUNKS, N_CHUNKS)], igv)

    def issue_gathers(c, buf, pa, pb, sem):
        pltpu.async_copy(v2_hbm.at[srcv.at[c]], buf, sem)
        pltpu.async_copy(p1_hbm.at[ilv.at[c]], pa, sem)
        pltpu.async_copy(p2_hbm.at[igv.at[c]], pb, sem)

    def wait_gathers(buf, pa, pb, sem):
        pltpu.make_async_copy(v2_hbm.at[pl.ds(0, CHUNK)], buf, sem).wait()
        pltpu.make_async_copy(p1_hbm.at[pl.ds(0, CHUNK)], pa, sem).wait()
        pltpu.make_async_copy(p2_hbm.at[pl.ds(0, CHUNK)], pb, sem).wait()

    def add_pe(buf, pa, pb):
        def tok_body(t, c):
            for j in range(HALF // 16):
                plsc.addupdate(buf.at[t, pl.ds(j * 16, 16)],
                               pa[t, pl.ds(j * 16, 16)])
            for j in range(HALF // 16):
                plsc.addupdate(buf.at[t, pl.ds(HALF + j * 16, 16)],
                               pb[t, pl.ds(j * 16, 16)])
            return c

        lax.fori_loop(0, CHUNK, tok_body, 0)

    issue_gathers(0, buf0, pa0, pb0, sg0)
    issue_gathers(1, buf1, pa1, pb1, sg1)

    def super_body(g, carry):
        c0 = 2 * g
        c1 = 2 * g + 1
        base0 = wid * TOK_PER_W + c0 * CHUNK
        base1 = wid * TOK_PER_W + c1 * CHUNK
        wait_gathers(buf0, pa0, pb0, sg0)
        add_pe(buf0, pa0, pb0)
        pltpu.async_copy(buf0, out_hbm.at[pl.ds(base0, CHUNK)], so0)
        wait_gathers(buf1, pa1, pb1, sg1)
        add_pe(buf1, pa1, pb1)
        pltpu.async_copy(buf1, out_hbm.at[pl.ds(base1, CHUNK)], so1)
        pltpu.make_async_copy(buf0, out_hbm.at[pl.ds(0, CHUNK)], so0).wait()

        @pl.when(c0 + 2 < N_CHUNKS)
        def _():
            issue_gathers(c0 + 2, buf0, pa0, pb0, sg0)

        pltpu.make_async_copy(buf1, out_hbm.at[pl.ds(0, CHUNK)], so1).wait()

        @pl.when(c1 + 2 < N_CHUNKS)
        def _():
            issue_gathers(c1 + 2, buf1, pa1, pb1, sg1)

        return carry

    lax.fori_loop(0, N_CHUNKS // 2, super_body, 0)


def kernel(src, frac, cbfv, W, b, emb_scaler, pos_scaler, pos_scaler_log):
    pe = _pe_table()
    frac2 = frac.reshape(N // 128, 128)
    cbfv_pad = jnp.zeros((_VPAD, _KPAD), jnp.float32).at[:VOCAB, :FEAT].set(cbfv)
    w_pad = jnp.zeros((_KPAD, D_MODEL), jnp.float32).at[:FEAT, :].set(W)
    brow = b.reshape(1, D_MODEL)
    es = jnp.broadcast_to(emb_scaler.reshape(1, 1), (1, D_MODEL))
    ps = jnp.broadcast_to(pos_scaler.reshape(1, 1), (1, HALF))
    pls = jnp.broadcast_to(pos_scaler_log.reshape(1, 1), (1, HALF))

    v2, il, ig, p1, p2 = pl.pallas_call(
        _prep,
        out_shape=[
            jax.ShapeDtypeStruct((_VPAD, D_MODEL), jnp.float32),
            jax.ShapeDtypeStruct((N // 128, 128), jnp.int32),
            jax.ShapeDtypeStruct((N // 128, 128), jnp.int32),
            jax.ShapeDtypeStruct((RES, HALF), jnp.float32),
            jax.ShapeDtypeStruct((RES, HALF), jnp.float32),
        ],
    )(frac2, cbfv_pad, w_pad, brow, es, ps, pls, pe)

    mesh = plsc.VectorSubcoreMesh(core_axis_name="c", subcore_axis_name="s")
    sc = functools.partial(
        pl.kernel,
        mesh=mesh,
        out_type=jax.ShapeDtypeStruct((N, D_MODEL), jnp.float32),
        scratch_types=[
            pltpu.VMEM((N_CHUNKS, CHUNK), jnp.int32),
            pltpu.VMEM((N_CHUNKS, CHUNK), jnp.int32),
            pltpu.VMEM((N_CHUNKS, CHUNK), jnp.int32),
            pltpu.VMEM((CHUNK, D_MODEL), jnp.float32),
            pltpu.VMEM((CHUNK, HALF), jnp.float32),
            pltpu.VMEM((CHUNK, HALF), jnp.float32),
            pltpu.VMEM((CHUNK, D_MODEL), jnp.float32),
            pltpu.VMEM((CHUNK, HALF), jnp.float32),
            pltpu.VMEM((CHUNK, HALF), jnp.float32),
            pltpu.SemaphoreType.DMA,
            pltpu.SemaphoreType.DMA,
            pltpu.SemaphoreType.DMA,
            pltpu.SemaphoreType.DMA,
        ],
    )(_sc_body)

    out = sc(v2, p1, p2, src.reshape(N // CHUNK, CHUNK),
             il.reshape(N // CHUNK, CHUNK), ig.reshape(N // CHUNK, CHUNK))
    return out.reshape(B, T, D_MODEL)


# speedup vs baseline: 1.1038x; 1.0051x over previous
import functools

import jax
import jax.numpy as jnp
from jax import lax
from jax.experimental import pallas as pl
from jax.experimental.pallas import tpu as pltpu
from jax.experimental.pallas import tpu_sc as plsc

D_MODEL = 512
HALF = D_MODEL // 2
RES = 5000
FEAT = 200
VOCAB = 119
B, T = 4096, 20
N = B * T

_NC, _NS = 2, 16
NW = _NC * _NS
TOK_PER_W = N // NW
CHUNK = 40
N_CHUNKS = TOK_PER_W // CHUNK

_VPAD = 120
_KPAD = 256


def _pe_table():
    x = jnp.arange(RES, dtype=jnp.float32).reshape(RES, 1)
    fr = jnp.tile(jnp.arange(HALF, dtype=jnp.float32).reshape(1, HALF), (RES, 1))
    pe = jnp.zeros((RES, HALF), dtype=jnp.float32)
    pe = pe.at[:, 0::2].set(jnp.sin(x / jnp.power(50.0, 2.0 * fr[:, 0::2] / HALF)))
    pe = pe.at[:, 1::2].set(jnp.cos(x / jnp.power(50.0, 2.0 * fr[:, 1::2] / HALF)))
    return pe


def _prep(frac_ref, cbfv_ref, w_ref, brow_ref, es_ref, ps_ref, pls_ref, pe_ref,
          v2_ref, il_ref, ig_ref, p1_ref, p2_ref):
    x = jnp.dot(cbfv_ref[...], w_ref[...], preferred_element_type=jnp.float32)
    v2_ref[...] = (x + brow_ref[...]) * (2.0 ** es_ref[...])
    p1_ref[...] = pe_ref[...] * (2.0 ** ((1.0 - ps_ref[...]) ** 2))
    p2_ref[...] = pe_ref[...] * (2.0 ** ((1.0 - pls_ref[...]) ** 2))
    f = frac_ref[...]
    xf = jnp.maximum(f, 1.0 / RES)
    il_ref[...] = jnp.round(xf * RES).astype(jnp.int32) - 1
    lg = jnp.log2(f)
    xl = 0.0025 * lg * lg
    xl = jnp.maximum(jnp.minimum(xl, 1.0), 1.0 / RES)
    ig_ref[...] = jnp.round(xl * RES).astype(jnp.int32) - 1


def _sc_body(v2_hbm, p1_hbm, p2_hbm, src_hbm, il_hbm, ig_hbm, out_hbm,
             srcv, ilv, igv, buf0, pa0, pb0, buf1, pa1, pb1,
             sg0, sg1, so0, so1):
    wid = lax.axis_index("s") * _NC + lax.axis_index("c")

    pltpu.sync_copy(src_hbm.at[pl.ds(wid * N_CHUNKS, N_CHUNKS)], srcv)
    pltpu.sync_copy(il_hbm.at[pl.ds(wid * N_CHUNKS, N_CHUNKS)], ilv)
    pltpu.sync_copy(ig_hbm.at[pl.ds(wid * N_CHUNKS, N_CHUNKS)], igv)

    def issue_gathers(c, buf, pa, pb, sem):
        pltpu.async_copy(v2_hbm.at[srcv.at[c]], buf, sem)
        pltpu.async_copy(p1_hbm.at[ilv.at[c]], pa, sem)
        pltpu.async_copy(p2_hbm.at[igv.at[c]], pb, sem)

    def wait_gathers(buf, pa, pb, sem):
        pltpu.make_async_copy(v2_hbm.at[pl.ds(0, CHUNK)], buf, sem).wait()
        pltpu.make_async_copy(p1_hbm.at[pl.ds(0, CHUNK)], pa, sem).wait()
        pltpu.make_async_copy(p2_hbm.at[pl.ds(0, CHUNK)], pb, sem).wait()

    def add_pe(buf, pa, pb):
        @plsc.parallel_loop(0, CHUNK, unroll=2)
        def tok_body(t):
            va = [pa[t, pl.ds(j * 16, 16)] for j in range(HALF // 16)]
            vb = [pb[t, pl.ds(j * 16, 16)] for j in range(HALF // 16)]
            for j in range(HALF // 16):
                plsc.addupdate(buf.at[t, pl.ds(j * 16, 16)], va[j])
            for j in range(HALF // 16):
                plsc.addupdate(buf.at[t, pl.ds(HALF + j * 16, 16)], vb[j])

    issue_gathers(0, buf0, pa0, pb0, sg0)
    issue_gathers(1, buf1, pa1, pb1, sg1)

    def super_body(g, carry):
        c0 = 2 * g
        c1 = 2 * g + 1
        base0 = wid * TOK_PER_W + c0 * CHUNK
        base1 = wid * TOK_PER_W + c1 * CHUNK
        wait_gathers(buf0, pa0, pb0, sg0)
        add_pe(buf0, pa0, pb0)
        pltpu.async_copy(buf0, out_hbm.at[pl.ds(base0, CHUNK)], so0)
        wait_gathers(buf1, pa1, pb1, sg1)
        add_pe(buf1, pa1, pb1)
        pltpu.async_copy(buf1, out_hbm.at[pl.ds(base1, CHUNK)], so1)
        pltpu.make_async_copy(buf0, out_hbm.at[pl.ds(0, CHUNK)], so0).wait()

        @pl.when(c0 + 2 < N_CHUNKS)
        def _():
            issue_gathers(c0 + 2, buf0, pa0, pb0, sg0)

        pltpu.make_async_copy(buf1, out_hbm.at[pl.ds(0, CHUNK)], so1).wait()

        @pl.when(c1 + 2 < N_CHUNKS)
        def _():
            issue_gathers(c1 + 2, buf1, pa1, pb1, sg1)

        return carry

    lax.fori_loop(0, N_CHUNKS // 2, super_body, 0)


def kernel(src, frac, cbfv, W, b, emb_scaler, pos_scaler, pos_scaler_log):
    pe = _pe_table()
    frac2 = frac.reshape(N // 128, 128)
    cbfv_pad = jnp.zeros((_VPAD, _KPAD), jnp.float32).at[:VOCAB, :FEAT].set(cbfv)
    w_pad = jnp.zeros((_KPAD, D_MODEL), jnp.float32).at[:FEAT, :].set(W)
    brow = b.reshape(1, D_MODEL)
    es = jnp.broadcast_to(emb_scaler.reshape(1, 1), (1, D_MODEL))
    ps = jnp.broadcast_to(pos_scaler.reshape(1, 1), (1, HALF))
    pls = jnp.broadcast_to(pos_scaler_log.reshape(1, 1), (1, HALF))

    v2, il, ig, p1, p2 = pl.pallas_call(
        _prep,
        out_shape=[
            jax.ShapeDtypeStruct((_VPAD, D_MODEL), jnp.float32),
            jax.ShapeDtypeStruct((N // 128, 128), jnp.int32),
            jax.ShapeDtypeStruct((N // 128, 128), jnp.int32),
            jax.ShapeDtypeStruct((RES, HALF), jnp.float32),
            jax.ShapeDtypeStruct((RES, HALF), jnp.float32),
        ],
    )(frac2, cbfv_pad, w_pad, brow, es, ps, pls, pe)

    mesh = plsc.VectorSubcoreMesh(core_axis_name="c", subcore_axis_name="s")
    sc = functools.partial(
        pl.kernel,
        mesh=mesh,
        out_type=jax.ShapeDtypeStruct((N, D_MODEL), jnp.float32),
        scratch_types=[
            pltpu.VMEM((N_CHUNKS, CHUNK), jnp.int32),
            pltpu.VMEM((N_CHUNKS, CHUNK), jnp.int32),
            pltpu.VMEM((N_CHUNKS, CHUNK), jnp.int32),
            pltpu.VMEM((CHUNK, D_MODEL), jnp.float32),
            pltpu.VMEM((CHUNK, HALF), jnp.float32),
            pltpu.VMEM((CHUNK, HALF), jnp.float32),
            pltpu.VMEM((CHUNK, D_MODEL), jnp.float32),
            pltpu.VMEM((CHUNK, HALF), jnp.float32),
            pltpu.VMEM((CHUNK, HALF), jnp.float32),
            pltpu.SemaphoreType.DMA,
            pltpu.SemaphoreType.DMA,
            pltpu.SemaphoreType.DMA,
            pltpu.SemaphoreType.DMA,
        ],
    )(_sc_body)

    out = sc(v2, p1, p2, src.reshape(N // CHUNK, CHUNK),
             il.reshape(N // CHUNK, CHUNK), ig.reshape(N // CHUNK, CHUNK))
    return out.reshape(B, T, D_MODEL)
